# baseline (speedup 1.0000x reference)
import functools

import jax
import jax.numpy as jnp
from jax.experimental import pallas as pl
from jax.experimental.pallas import tpu as pltpu

_K = 8192
_R = 256
_A = 16
_B = 3
_BK = 1024


def _body(invt_ref, l_ref, w_ref, atoms_ref, o_ref):
    invt = invt_ref[0, 0]
    e = jnp.exp(l_ref[...] * invt)
    w = w_ref[...]
    m = None
    for b in range(_B):
        eb = e[:, b * _A:(b + 1) * _A]
        sb = jnp.sum(eb, axis=1, keepdims=True)
        cb = w[:, b:b + 1] / sb
        mb = eb * cb
        m = mb if m is None else m + mb
    a_soft = jnp.tanh(atoms_ref[...] * invt)
    o_ref[...] = jnp.dot(m, a_soft, preferred_element_type=jnp.float32)


@functools.partial(jax.jit, static_argnames=("interpret",))
def kernel(atoms, combo_weights, combo_indices_logits, temperature, interpret=False):
    k, b, a = combo_indices_logits.shape
    r = atoms.shape[1]
    invt = (1.0 / jnp.maximum(jnp.asarray(temperature, jnp.float32), 0.1))
    invt = invt.reshape(1, 1)
    logits2 = combo_indices_logits.reshape(k, b * a)
    grid = (k // _BK,)
    return pl.pallas_call(
        _body,
        grid=grid,
        in_specs=[
            pl.BlockSpec((1, 1), lambda i: (0, 0), memory_space=pltpu.SMEM),
            pl.BlockSpec((_BK, b * a), lambda i: (i, 0)),
            pl.BlockSpec((_BK, b), lambda i: (i, 0)),
            pl.BlockSpec((a, r), lambda i: (0, 0)),
        ],
        out_specs=pl.BlockSpec((_BK, r), lambda i: (i, 0)),
        out_shape=jax.ShapeDtypeStruct((k, r), jnp.float32),
        interpret=interpret,
    )(invt, logits2, combo_weights, atoms)


# speedup vs baseline: 1.2722x; 1.2722x over previous
import functools

import jax
import jax.numpy as jnp
from jax import lax
from jax.experimental import pallas as pl
from jax.experimental.pallas import tpu as pltpu

_A = 16
_B = 3
_BK = 1024


def _body(invt_ref, l_ref, w_ref, atoms_ref, sel_ref, o_ref):
    invt = invt_ref[0, 0]
    e = jnp.exp(l_ref[...] * invt)
    sel = sel_ref[...]
    s3 = lax.dot_general(e, sel, (((1,), (0,)), ((), ())),
                         preferred_element_type=jnp.float32)
    rs = w_ref[...] / s3
    crep = lax.dot_general(rs, sel, (((1,), (1,)), ((), ())),
                           preferred_element_type=jnp.float32)
    q = e * crep
    a_soft = jnp.tanh(atoms_ref[...] * invt)
    a_tiled = jnp.concatenate([a_soft] * _B, axis=0)
    o_ref[...] = jnp.dot(q, a_tiled, preferred_element_type=jnp.float32)


@functools.partial(jax.jit, static_argnames=("interpret",))
def kernel(atoms, combo_weights, combo_indices_logits, temperature, interpret=False):
    k, b, a = combo_indices_logits.shape
    r = atoms.shape[1]
    invt = (1.0 / jnp.maximum(jnp.asarray(temperature, jnp.float32), 0.1))
    invt = invt.reshape(1, 1)
    logits2 = combo_indices_logits.reshape(k, b * a)
    sel = jnp.repeat(jnp.eye(b, dtype=jnp.float32), a, axis=0)
    grid = (k // _BK,)
    return pl.pallas_call(
        _body,
        grid=grid,
        in_specs=[
            pl.BlockSpec((1, 1), lambda i: (0, 0), memory_space=pltpu.SMEM),
            pl.BlockSpec((_BK, b * a), lambda i: (i, 0)),
            pl.BlockSpec((_BK, b), lambda i: (i, 0)),
            pl.BlockSpec((a, r), lambda i: (0, 0)),
            pl.BlockSpec((b * a, b), lambda i: (0, 0)),
        ],
        out_specs=pl.BlockSpec((_BK, r), lambda i: (i, 0)),
        out_shape=jax.ShapeDtypeStruct((k, r), jnp.float32),
        interpret=interpret,
    )(invt, logits2, combo_weights, atoms, sel)


# speedup vs baseline: 2.6385x; 2.0739x over previous
import functools

import jax
import jax.numpy as jnp
from jax import lax
from jax.experimental import pallas as pl
from jax.experimental.pallas import tpu as pltpu

_A = 16
_B = 3
_BK = 1024


def _body(invt_ref, lT_ref, wT_ref, atoms_ref, o_ref):
    invt = invt_ref[0, 0]
    e = jnp.exp(lT_ref[...] * invt)
    e3 = e.reshape(_B, _A, e.shape[-1])
    s = jnp.sum(e3, axis=1, keepdims=True)
    c = wT_ref[...].reshape(_B, 1, -1) / s
    m = jnp.sum(e3 * c, axis=0)
    a_soft = jnp.tanh(atoms_ref[...] * invt)
    o_ref[...] = lax.dot_general(
        m, a_soft, dimension_numbers=(((0,), (0,)), ((), ())),
        preferred_element_type=jnp.float32)


@functools.partial(jax.jit, static_argnames=("interpret",))
def kernel(atoms, combo_weights, combo_indices_logits, temperature, interpret=False):
    k, b, a = combo_indices_logits.shape
    r = atoms.shape[1]
    invt = (1.0 / jnp.maximum(jnp.asarray(temperature, jnp.float32), 0.1))
    invt = invt.reshape(1, 1)
    lT = combo_indices_logits.transpose(1, 2, 0).reshape(b * a, k)
    wT = combo_weights.T
    grid = (k // _BK,)
    return pl.pallas_call(
        _body,
        grid=grid,
        in_specs=[
            pl.BlockSpec((1, 1), lambda i: (0, 0), memory_space=pltpu.SMEM),
            pl.BlockSpec((b * a, _BK), lambda i: (0, i)),
            pl.BlockSpec((b, _BK), lambda i: (0, i)),
            pl.BlockSpec((a, r), lambda i: (0, 0)),
        ],
        out_specs=pl.BlockSpec((_BK, r), lambda i: (i, 0)),
        out_shape=jax.ShapeDtypeStruct((k, r), jnp.float32),
        interpret=interpret,
    )(invt, lT, wT, atoms)
